# Initial kernel scaffold; baseline (speedup 1.0000x reference)
import jax
import jax.numpy as jnp
from jax import lax
from jax.experimental import pallas as pl
from jax.experimental.pallas import tpu as pltpu
from jax.experimental.pallas import tpu_sc as plsc

B, TG, M = 16, 2048, 4
E2 = 128
IN = M * 2 * E2
H = IN // 2
K_OUT = 1024
NC, NS = 2, 16
NW = NC * NS
R = B * TG * M * 2
RPW = R // NW
CH = 128
NCHUNK = RPW // CH
SEL = B * K_OUT
SPW = SEL // NW
PW = 16

_VMESH = plsc.VectorSubcoreMesh(core_axis_name="c", subcore_axis_name="s")


def _worker_id():
    return lax.axis_index("s") * NC + lax.axis_index("c")


def _sc_gather_body(table_hbm, idx_hbm, out_hbm, idx_v, rows_v, sem):
    wid = _worker_id()
    base = wid * NCHUNK
    pltpu.sync_copy(idx_hbm.at[pl.ds(base, NCHUNK)], idx_v)

    def body(j, carry):
        pltpu.async_copy(table_hbm.at[idx_v.at[j]], rows_v, sem).wait()
        pltpu.sync_copy(rows_v, out_hbm.at[pl.ds((base + j) * CH, CH)])
        return carry

    lax.fori_loop(0, NCHUNK, body, 0)


def _sc_gather(table, idx2d):
    k = pl.kernel(
        _sc_gather_body,
        out_type=jax.ShapeDtypeStruct((R, E2), jnp.float32),
        mesh=_VMESH,
        scratch_types=[
            pltpu.VMEM((NCHUNK, CH), jnp.int32),
            pltpu.VMEM((CH, E2), jnp.float32),
            pltpu.SemaphoreType.DMA,
        ],
    )
    return k(table, idx2d)


def _mlp_body(flat_ref, body0_ref, maskf_ref, w1_ref, b1_ref, w2_ref, b2_ref,
              out_ref):
    a = (body0_ref[...] != 0).astype(jnp.float32)
    acc = jnp.zeros((TG, H), jnp.float32) + b1_ref[...]
    for m in range(M):
        xm = flat_ref[:, 2 * E2 * m:2 * E2 * (m + 1)] * a[:, m:m + 1]
        acc = acc + jnp.dot(xm, w1_ref[2 * E2 * m:2 * E2 * (m + 1), :],
                            preferred_element_type=jnp.float32)
    h = jnp.maximum(acc, 0.0)
    s = jnp.dot(h, w2_ref[...], preferred_element_type=jnp.float32)
    s = s + b2_ref[...]
    out_ref[...] = s + (maskf_ref[...] - 1.0) * 1000000000.0


def _mlp(flat, body0, maskf, W1, b1, W2, b2):
    return pl.pallas_call(
        _mlp_body,
        grid=(B,),
        in_specs=[
            pl.BlockSpec((TG, IN), lambda i: (i, 0)),
            pl.BlockSpec((TG, M), lambda i: (i, 0)),
            pl.BlockSpec((TG, 1), lambda i: (i, 0)),
            pl.BlockSpec((IN, H), lambda i: (0, 0)),
            pl.BlockSpec((1, H), lambda i: (0, 0)),
            pl.BlockSpec((H, 1), lambda i: (0, 0)),
            pl.BlockSpec((1, 1), lambda i: (0, 0)),
        ],
        out_specs=pl.BlockSpec((TG, 1), lambda i: (i, 0)),
        out_shape=jax.ShapeDtypeStruct((B * TG, 1), jnp.float32),
    )(flat, body0, maskf, W1, b1, W2, b2)


def _topk_body(s_ref, score_ref, gidx_ref):
    key = s_ref[...]
    l = lax.broadcasted_iota(jnp.int32, (B, TG), 1)
    idx = l
    for k in range(1, 12):
        for j in range(k - 1, -1, -1):
            d = 1 << j
            partner_hi = (l & d) != 0
            pk = jnp.where(partner_hi, pltpu.roll(key, d, 1),
                           pltpu.roll(key, -d, 1))
            pi = jnp.where(partner_hi, pltpu.roll(idx, d, 1),
                           pltpu.roll(idx, -d, 1))
            dir_desc = (l & (1 << k)) == 0
            keep_max = jnp.logical_xor(dir_desc, partner_hi)
            i_larger = (key > pk) | ((key == pk) & (idx < pi))
            take_self = keep_max == i_larger
            key = jnp.where(take_self, key, pk)
            idx = jnp.where(take_self, idx, pi)
    row = lax.broadcasted_iota(jnp.int32, (B, K_OUT), 0)
    score_ref[...] = key[:, :K_OUT]
    gidx_ref[...] = idx[:, :K_OUT] + TG * row


def _topk(scores):
    return pl.pallas_call(
        _topk_body,
        out_shape=(
            jax.ShapeDtypeStruct((B, K_OUT), jnp.float32),
            jax.ShapeDtypeStruct((B, K_OUT), jnp.int32),
        ),
    )(scores)


def _sc_select_body(pay_hbm, idx_hbm, out_hbm, idx_v, rows_v, sem):
    wid = _worker_id()
    nch = SPW // CH
    base = wid * nch
    pltpu.sync_copy(idx_hbm.at[pl.ds(base, nch)], idx_v)

    def body(j, carry):
        pltpu.async_copy(pay_hbm.at[idx_v.at[j]], rows_v, sem).wait()
        pltpu.sync_copy(rows_v, out_hbm.at[pl.ds((base + j) * CH, CH)])
        return carry

    lax.fori_loop(0, SPW // CH, body, 0)


def _sc_select(payload, idx2d):
    k = pl.kernel(
        _sc_select_body,
        out_type=jax.ShapeDtypeStruct((SEL, PW), jnp.int32),
        mesh=_VMESH,
        scratch_types=[
            pltpu.VMEM((SPW // CH, CH), jnp.int32),
            pltpu.VMEM((CH, PW), jnp.int32),
            pltpu.SemaphoreType.DMA,
        ],
    )
    return k(payload, idx2d)


def kernel(body, mask, rule_idx, ent_table, W1, b1, W2, b2):
    gidx = body[..., 1:3].reshape(R // CH, CH)
    flat = _sc_gather(ent_table, gidx)

    body0 = body[..., 0].reshape(B * TG, M)
    maskf = mask.astype(jnp.float32).reshape(B * TG, 1)
    scores = _mlp(flat.reshape(B * TG, IN), body0, maskf, W1,
                  b1.reshape(1, H), W2, b2.reshape(1, 1))

    top_scores, sel_idx = _topk(scores.reshape(B, TG))

    payload = jnp.concatenate(
        [body.reshape(B * TG, 12),
         rule_idx.reshape(B * TG, 1),
         mask.astype(jnp.int32).reshape(B * TG, 1),
         jnp.zeros((B * TG, 2), jnp.int32)], axis=1)
    sel = _sc_select(payload, sel_idx.reshape(SEL // CH, CH))
    sel = sel.reshape(B, K_OUT, PW)

    body_out = sel[..., :12].reshape(B, K_OUT, M, 3)
    rule_out = sel[..., 12]
    mask_out = sel[..., 13].astype(jnp.bool_)
    return body_out, mask_out, rule_out, top_scores


# speedup vs baseline: 1.4339x; 1.4339x over previous
import jax
import jax.numpy as jnp
from jax import lax
from jax.experimental import pallas as pl
from jax.experimental.pallas import tpu as pltpu
from jax.experimental.pallas import tpu_sc as plsc

B, TG, M = 16, 2048, 4
E2 = 128
IN = M * 2 * E2
H = IN // 2
K_OUT = 1024
NC, NS = 2, 16
NW = NC * NS
R = B * TG * M * 2
RPW = R // NW
CH = 128
NCHUNK = RPW // CH
SEL = B * K_OUT
SPW = SEL // NW
PW = 16

def _vmesh():
    return plsc.VectorSubcoreMesh(core_axis_name="c", subcore_axis_name="s",
                                  num_cores=NC, num_subcores=NS)


def _worker_id():
    return lax.axis_index("s") * NC + lax.axis_index("c")


def _sc_gather_body(table_hbm, idx_hbm, out_hbm, idx_v, rows_v, sem):
    wid = _worker_id()
    base = wid * NCHUNK
    pltpu.sync_copy(idx_hbm.at[pl.ds(base, NCHUNK)], idx_v)

    def body(j, carry):
        pltpu.async_copy(table_hbm.at[idx_v.at[j]], rows_v, sem).wait()
        pltpu.sync_copy(rows_v, out_hbm.at[pl.ds((base + j) * CH, CH)])
        return carry

    lax.fori_loop(0, NCHUNK, body, 0)


def _sc_gather(table, idx2d):
    k = pl.kernel(
        _sc_gather_body,
        out_type=jax.ShapeDtypeStruct((R, E2), jnp.float32),
        mesh=_vmesh(),
        scratch_types=[
            pltpu.VMEM((NCHUNK, CH), jnp.int32),
            pltpu.VMEM((CH, E2), jnp.float32),
            pltpu.SemaphoreType.DMA,
        ],
    )
    return k(table, idx2d)


def _mlp_body(flat_ref, body0_ref, maskf_ref, w1_ref, b1_ref, w2_ref, b2_ref,
              out_ref):
    a = (body0_ref[...] != 0).astype(jnp.float32)
    x = jnp.concatenate(
        [flat_ref[:, 2 * E2 * m:2 * E2 * (m + 1)] * a[:, m:m + 1]
         for m in range(M)], axis=1)
    acc = jnp.dot(x, w1_ref[...],
                  preferred_element_type=jnp.float32) + b1_ref[...]
    h = jnp.maximum(acc, 0.0)
    s = jnp.dot(h, w2_ref[...], preferred_element_type=jnp.float32)
    s = s + b2_ref[...]
    out_ref[...] = s + (maskf_ref[...] - 1.0) * 1000000000.0


def _mlp(flat, body0, maskf, W1, b1, W2, b2):
    return pl.pallas_call(
        _mlp_body,
        grid=(B,),
        in_specs=[
            pl.BlockSpec((TG, IN), lambda i: (i, 0)),
            pl.BlockSpec((TG, M), lambda i: (i, 0)),
            pl.BlockSpec((TG, 1), lambda i: (i, 0)),
            pl.BlockSpec((IN, H), lambda i: (0, 0)),
            pl.BlockSpec((1, H), lambda i: (0, 0)),
            pl.BlockSpec((H, 1), lambda i: (0, 0)),
            pl.BlockSpec((1, 1), lambda i: (0, 0)),
        ],
        out_specs=pl.BlockSpec((TG, 1), lambda i: (i, 0)),
        out_shape=jax.ShapeDtypeStruct((B * TG, 1), jnp.float32),
    )(flat, body0, maskf, W1, b1, W2, b2)


def _topk_body(s_ref, score_ref, gidx_ref):
    key = s_ref[...]
    l = lax.broadcasted_iota(jnp.int32, (B, TG), 1)
    idx = l
    for k in range(1, 12):
        for j in range(k - 1, -1, -1):
            d = 1 << j
            partner_hi = (l & d) != 0
            pk = jnp.where(partner_hi, pltpu.roll(key, d, 1),
                           pltpu.roll(key, TG - d, 1))
            pi = jnp.where(partner_hi, pltpu.roll(idx, d, 1),
                           pltpu.roll(idx, TG - d, 1))
            dir_desc = (l & (1 << k)) == 0
            keep_max = jnp.logical_xor(dir_desc, partner_hi)
            i_larger = (key > pk) | ((key == pk) & (idx < pi))
            take_self = keep_max == i_larger
            key = jnp.where(take_self, key, pk)
            idx = jnp.where(take_self, idx, pi)
    row = lax.broadcasted_iota(jnp.int32, (B, K_OUT), 0)
    score_ref[...] = key[:, :K_OUT]
    gidx_ref[...] = idx[:, :K_OUT] + TG * row


def _topk(scores):
    return pl.pallas_call(
        _topk_body,
        out_shape=(
            jax.ShapeDtypeStruct((B, K_OUT), jnp.float32),
            jax.ShapeDtypeStruct((B, K_OUT), jnp.int32),
        ),
    )(scores)


def _sc_select_body(pay_hbm, idx_hbm, out_hbm, idx_v, rows_v, sem):
    wid = _worker_id()
    nch = SPW // CH
    base = wid * nch
    pltpu.sync_copy(idx_hbm.at[pl.ds(base, nch)], idx_v)

    def body(j, carry):
        pltpu.async_copy(pay_hbm.at[idx_v.at[j]], rows_v, sem).wait()
        pltpu.sync_copy(rows_v, out_hbm.at[pl.ds((base + j) * CH, CH)])
        return carry

    lax.fori_loop(0, SPW // CH, body, 0)


def _sc_select(payload, idx2d):
    k = pl.kernel(
        _sc_select_body,
        out_type=jax.ShapeDtypeStruct((SEL, PW), jnp.int32),
        mesh=_vmesh(),
        scratch_types=[
            pltpu.VMEM((SPW // CH, CH), jnp.int32),
            pltpu.VMEM((CH, PW), jnp.int32),
            pltpu.SemaphoreType.DMA,
        ],
        compiler_params=pltpu.CompilerParams(use_tc_tiling_on_sc=False),
    )
    return k(payload, idx2d)


def kernel(body, mask, rule_idx, ent_table, W1, b1, W2, b2):
    gidx = body[..., 1:3].reshape(R // CH, CH)
    flat = _sc_gather(ent_table, gidx)

    body0 = body[..., 0].reshape(B * TG, M)
    maskf = mask.astype(jnp.float32).reshape(B * TG, 1)
    scores = _mlp(flat.reshape(B * TG, IN), body0, maskf, W1,
                  b1.reshape(1, H), W2, b2.reshape(1, 1))

    top_scores, sel_idx = _topk(scores.reshape(B, TG))

    payload = jnp.concatenate(
        [body.reshape(B * TG, 12),
         rule_idx.reshape(B * TG, 1),
         mask.astype(jnp.int32).reshape(B * TG, 1),
         jnp.zeros((B * TG, 2), jnp.int32)], axis=1)
    sel = _sc_select(payload, sel_idx.reshape(SEL // CH, CH))
    sel = sel.reshape(B, K_OUT, PW)

    body_out = sel[..., :12].reshape(B, K_OUT, M, 3)
    rule_out = sel[..., 12]
    mask_out = sel[..., 13].astype(jnp.bool_)
    return body_out, mask_out, rule_out, top_scores


# speedup vs baseline: 1.8993x; 1.3246x over previous
import jax
import jax.numpy as jnp
from jax import lax
from jax.experimental import pallas as pl
from jax.experimental.pallas import tpu as pltpu
from jax.experimental.pallas import tpu_sc as plsc

B, TG, M = 16, 2048, 4
E2 = 128
IN = M * 2 * E2
H = IN // 2
K_OUT = 1024
NC, NS = 2, 16
NW = NC * NS
R = B * TG * M * 2
RPW = R // NW
CH = 128
NCHUNK = RPW // CH
SEL = B * K_OUT
SPW = SEL // NW
PW = 16

def _vmesh():
    return plsc.VectorSubcoreMesh(core_axis_name="c", subcore_axis_name="s",
                                  num_cores=NC, num_subcores=NS)


def _worker_id():
    return lax.axis_index("s") * NC + lax.axis_index("c")


def _sc_gather_body(table_hbm, idx_hbm, out_hbm, idx_v, rows_v, sem):
    wid = _worker_id()
    base = wid * NCHUNK
    pltpu.sync_copy(idx_hbm.at[pl.ds(base, NCHUNK)], idx_v)
    gpc = CH // 8

    def body(j, carry):
        pltpu.async_copy(table_hbm.at[idx_v.at[j]], rows_v, sem).wait()
        pltpu.sync_copy(rows_v.reshape(gpc, IN),
                        out_hbm.at[pl.ds((base + j) * gpc, gpc)])
        return carry

    lax.fori_loop(0, NCHUNK, body, 0)


def _sc_gather(table, idx2d):
    k = pl.kernel(
        _sc_gather_body,
        out_type=jax.ShapeDtypeStruct((B * TG, IN), jnp.float32),
        mesh=_vmesh(),
        scratch_types=[
            pltpu.VMEM((NCHUNK, CH), jnp.int32),
            pltpu.VMEM((CH, E2), jnp.float32),
            pltpu.SemaphoreType.DMA,
        ],
    )
    return k(table, idx2d)


def _mlp_body(flat_ref, body0_ref, maskf_ref, w1_ref, b1_ref, w2_ref, b2_ref,
              out_ref):
    a = (body0_ref[...] != 0).astype(jnp.float32)
    x = jnp.concatenate(
        [flat_ref[:, 2 * E2 * m:2 * E2 * (m + 1)] * a[:, m:m + 1]
         for m in range(M)], axis=1)
    acc = jnp.dot(x, w1_ref[...],
                  preferred_element_type=jnp.float32) + b1_ref[...]
    h = jnp.maximum(acc, 0.0)
    s = jnp.dot(h, w2_ref[...], preferred_element_type=jnp.float32)
    s = s + b2_ref[...]
    out_ref[...] = s + (maskf_ref[...] - 1.0) * 1000000000.0


def _mlp(flat, body0, maskf, W1, b1, W2, b2):
    return pl.pallas_call(
        _mlp_body,
        grid=(B,),
        in_specs=[
            pl.BlockSpec((TG, IN), lambda i: (i, 0)),
            pl.BlockSpec((TG, M), lambda i: (i, 0)),
            pl.BlockSpec((TG, 1), lambda i: (i, 0)),
            pl.BlockSpec((IN, H), lambda i: (0, 0)),
            pl.BlockSpec((1, H), lambda i: (0, 0)),
            pl.BlockSpec((H, 1), lambda i: (0, 0)),
            pl.BlockSpec((1, 1), lambda i: (0, 0)),
        ],
        out_specs=pl.BlockSpec((TG, 1), lambda i: (i, 0)),
        out_shape=jax.ShapeDtypeStruct((B * TG, 1), jnp.float32),
    )(flat, body0, maskf, W1, b1, W2, b2)


def _topk_body(s_ref, score_ref, gidx_ref):
    key = s_ref[...]
    l = lax.broadcasted_iota(jnp.int32, (B, TG), 1)
    idx = l
    for k in range(1, 12):
        for j in range(k - 1, -1, -1):
            d = 1 << j
            partner_hi = (l & d) != 0
            pk = jnp.where(partner_hi, pltpu.roll(key, d, 1),
                           pltpu.roll(key, TG - d, 1))
            pi = jnp.where(partner_hi, pltpu.roll(idx, d, 1),
                           pltpu.roll(idx, TG - d, 1))
            dir_desc = (l & (1 << k)) == 0
            keep_max = jnp.logical_xor(dir_desc, partner_hi)
            i_larger = (key > pk) | ((key == pk) & (idx < pi))
            take_self = keep_max == i_larger
            key = jnp.where(take_self, key, pk)
            idx = jnp.where(take_self, idx, pi)
    row = lax.broadcasted_iota(jnp.int32, (B, K_OUT), 0)
    score_ref[...] = key[:, :K_OUT]
    gidx_ref[...] = idx[:, :K_OUT] + TG * row


def _topk(scores):
    return pl.pallas_call(
        _topk_body,
        out_shape=(
            jax.ShapeDtypeStruct((B, K_OUT), jnp.float32),
            jax.ShapeDtypeStruct((B, K_OUT), jnp.int32),
        ),
    )(scores)


def _sc_select_body(pay_hbm, idx_hbm, out_hbm, idx_v, rows_v, sem):
    wid = _worker_id()
    nch = SPW // CH
    base = wid * nch
    pltpu.sync_copy(idx_hbm.at[pl.ds(base, nch)], idx_v)

    def body(j, carry):
        pltpu.async_copy(pay_hbm.at[idx_v.at[j]], rows_v, sem).wait()
        pltpu.sync_copy(rows_v, out_hbm.at[pl.ds((base + j) * CH, CH)])
        return carry

    lax.fori_loop(0, SPW // CH, body, 0)


def _sc_select(payload, idx2d):
    k = pl.kernel(
        _sc_select_body,
        out_type=jax.ShapeDtypeStruct((SEL, PW), jnp.int32),
        mesh=_vmesh(),
        scratch_types=[
            pltpu.VMEM((SPW // CH, CH), jnp.int32),
            pltpu.VMEM((CH, PW), jnp.int32),
            pltpu.SemaphoreType.DMA,
        ],
        compiler_params=pltpu.CompilerParams(use_tc_tiling_on_sc=False),
    )
    return k(payload, idx2d)


def kernel(body, mask, rule_idx, ent_table, W1, b1, W2, b2):
    gidx = body[..., 1:3].reshape(R // CH, CH)
    flat = _sc_gather(ent_table, gidx)

    body0 = body[..., 0].reshape(B * TG, M)
    maskf = mask.astype(jnp.float32).reshape(B * TG, 1)
    scores = _mlp(flat, body0, maskf, W1,
                  b1.reshape(1, H), W2, b2.reshape(1, 1))

    top_scores, sel_idx = _topk(scores.reshape(B, TG))

    payload = jnp.concatenate(
        [body.reshape(B * TG, 12),
         rule_idx.reshape(B * TG, 1),
         mask.astype(jnp.int32).reshape(B * TG, 1),
         jnp.zeros((B * TG, 2), jnp.int32)], axis=1)
    sel = _sc_select(payload, sel_idx.reshape(SEL // CH, CH))
    sel = sel.reshape(B, K_OUT, PW)

    body_out = sel[..., :12].reshape(B, K_OUT, M, 3)
    rule_out = sel[..., 12]
    mask_out = sel[..., 13].astype(jnp.bool_)
    return body_out, mask_out, rule_out, top_scores


# speedup vs baseline: 2.6221x; 1.3805x over previous
import jax
import jax.numpy as jnp
from jax import lax
from jax.experimental import pallas as pl
from jax.experimental.pallas import tpu as pltpu
from jax.experimental.pallas import tpu_sc as plsc

B, TG, M = 16, 2048, 4
E2 = 128
IN = M * 2 * E2
H = IN // 2
K_OUT = 1024
NC, NS = 2, 16
NW = NC * NS
R = B * TG * M * 2
RPW = R // NW
CH = 128
NCHUNK = RPW // CH
SEL = B * K_OUT
SPW = SEL // NW
PW = 16

def _vmesh():
    return plsc.VectorSubcoreMesh(core_axis_name="c", subcore_axis_name="s",
                                  num_cores=NC, num_subcores=NS)


def _worker_id():
    return lax.axis_index("s") * NC + lax.axis_index("c")


def _sc_gather_body(table_hbm, idx_hbm, out_hbm, idx_v, rows_v, sem):
    wid = _worker_id()
    base = wid * NCHUNK
    pltpu.sync_copy(idx_hbm.at[pl.ds(base, NCHUNK)], idx_v)

    def body(j, carry):
        pltpu.async_copy(table_hbm.at[idx_v.at[j]], rows_v, sem).wait()
        pltpu.sync_copy(rows_v, out_hbm.at[pl.ds((base + j) * CH, CH)])
        return carry

    lax.fori_loop(0, NCHUNK, body, 0)


def _sc_gather(table, idx2d):
    k = pl.kernel(
        _sc_gather_body,
        out_type=jax.ShapeDtypeStruct((R, E2), jnp.float32),
        mesh=_vmesh(),
        scratch_types=[
            pltpu.VMEM((NCHUNK, CH), jnp.int32),
            pltpu.VMEM((CH, E2), jnp.float32),
            pltpu.SemaphoreType.DMA,
        ],
    )
    return k(table, idx2d)


def _mlp_body(flat_ref, body0_ref, maskf_ref, w1_ref, b1_ref, w2_ref, b2_ref,
              out_ref):
    a = (body0_ref[...] != 0).astype(jnp.float32)
    x = jnp.concatenate(
        [flat_ref[s] * a[:, s // 2:s // 2 + 1] for s in range(2 * M)],
        axis=1)
    acc = jnp.dot(x, w1_ref[...],
                  preferred_element_type=jnp.float32) + b1_ref[...]
    h = jnp.maximum(acc, 0.0)
    s = jnp.dot(h, w2_ref[...], preferred_element_type=jnp.float32)
    s = s + b2_ref[...]
    s = s + (maskf_ref[...] - 1.0) * 1000000000.0
    out_ref[...] = s.reshape(1, 1, TG)


def _mlp(flat_s, body0, maskf, W1, b1, W2, b2):
    return pl.pallas_call(
        _mlp_body,
        grid=(B,),
        in_specs=[
            pl.BlockSpec((2 * M, TG, E2), lambda i: (0, i, 0)),
            pl.BlockSpec((TG, M), lambda i: (i, 0)),
            pl.BlockSpec((TG, 1), lambda i: (i, 0)),
            pl.BlockSpec((IN, H), lambda i: (0, 0)),
            pl.BlockSpec((1, H), lambda i: (0, 0)),
            pl.BlockSpec((H, 1), lambda i: (0, 0)),
            pl.BlockSpec((1, 1), lambda i: (0, 0)),
        ],
        out_specs=pl.BlockSpec((1, 1, TG), lambda i: (i, 0, 0)),
        out_shape=jax.ShapeDtypeStruct((B, 1, TG), jnp.float32),
    )(flat_s, body0, maskf, W1, b1, W2, b2)


def _topk_body(s_ref, score_ref, gidx_ref):
    key = s_ref[...]
    l = lax.broadcasted_iota(jnp.int32, (B, TG), 1)
    idx = l
    for k in range(1, 12):
        for j in range(k - 1, -1, -1):
            d = 1 << j
            partner_hi = (l & d) != 0
            pk = jnp.where(partner_hi, pltpu.roll(key, d, 1),
                           pltpu.roll(key, TG - d, 1))
            pi = jnp.where(partner_hi, pltpu.roll(idx, d, 1),
                           pltpu.roll(idx, TG - d, 1))
            dir_desc = (l & (1 << k)) == 0
            keep_max = jnp.logical_xor(dir_desc, partner_hi)
            i_larger = (key > pk) | ((key == pk) & (idx < pi))
            take_self = keep_max == i_larger
            key = jnp.where(take_self, key, pk)
            idx = jnp.where(take_self, idx, pi)
    row = lax.broadcasted_iota(jnp.int32, (B, K_OUT), 0)
    score_ref[...] = key[:, :K_OUT]
    gidx_ref[...] = idx[:, :K_OUT] + TG * row


def _topk(scores):
    return pl.pallas_call(
        _topk_body,
        out_shape=(
            jax.ShapeDtypeStruct((B, K_OUT), jnp.float32),
            jax.ShapeDtypeStruct((B, K_OUT), jnp.int32),
        ),
    )(scores)


def _sc_select_body(pay_hbm, idx_hbm, out_hbm, idx_v, rows_v, sem):
    wid = _worker_id()
    nch = SPW // CH
    base = wid * nch
    pltpu.sync_copy(idx_hbm.at[pl.ds(base, nch)], idx_v)

    def body(j, carry):
        pltpu.async_copy(pay_hbm.at[idx_v.at[j]], rows_v, sem).wait()
        pltpu.sync_copy(rows_v, out_hbm.at[pl.ds((base + j) * CH, CH)])
        return carry

    lax.fori_loop(0, SPW // CH, body, 0)


def _sc_select(payload, idx2d):
    k = pl.kernel(
        _sc_select_body,
        out_type=jax.ShapeDtypeStruct((SEL, PW), jnp.int32),
        mesh=_vmesh(),
        scratch_types=[
            pltpu.VMEM((SPW // CH, CH), jnp.int32),
            pltpu.VMEM((CH, PW), jnp.int32),
            pltpu.SemaphoreType.DMA,
        ],
        compiler_params=pltpu.CompilerParams(use_tc_tiling_on_sc=False),
    )
    return k(payload, idx2d)


def kernel(body, mask, rule_idx, ent_table, W1, b1, W2, b2):
    gidx = jnp.transpose(body[..., 1:3], (2, 3, 0, 1)).reshape(R // CH, CH)
    flat = _sc_gather(ent_table, gidx)
    flat_s = flat.reshape(2 * M, B * TG, E2)

    body0 = body[..., 0].reshape(B * TG, M)
    maskf = mask.astype(jnp.float32).reshape(B * TG, 1)
    scores = _mlp(flat_s, body0, maskf, W1,
                  b1.reshape(1, H), W2, b2.reshape(1, 1))

    top_scores, sel_idx = _topk(scores.reshape(B, TG))

    payload = jnp.concatenate(
        [body.reshape(B * TG, 12),
         rule_idx.reshape(B * TG, 1),
         mask.astype(jnp.int32).reshape(B * TG, 1),
         jnp.zeros((B * TG, 2), jnp.int32)], axis=1)
    sel = _sc_select(payload, sel_idx.reshape(SEL // CH, CH))
    sel = sel.reshape(B, K_OUT, PW)

    body_out = sel[..., :12].reshape(B, K_OUT, M, 3)
    rule_out = sel[..., 12]
    mask_out = sel[..., 13].astype(jnp.bool_)
    return body_out, mask_out, rule_out, top_scores


# speedup vs baseline: 2.7025x; 1.0307x over previous
import jax
import jax.numpy as jnp
from jax import lax
from jax.experimental import pallas as pl
from jax.experimental.pallas import tpu as pltpu
from jax.experimental.pallas import tpu_sc as plsc

B, TG, M = 16, 2048, 4
E2 = 128
IN = M * 2 * E2
H = IN // 2
K_OUT = 1024
NC, NS = 2, 16
NW = NC * NS
R = B * TG * M * 2
RPW = R // NW
CH = 128
NCHUNK = RPW // CH
SEL = B * K_OUT
SPW = SEL // NW
PW = 16

def _vmesh():
    return plsc.VectorSubcoreMesh(core_axis_name="c", subcore_axis_name="s",
                                  num_cores=NC, num_subcores=NS)


def _worker_id():
    return lax.axis_index("s") * NC + lax.axis_index("c")


NCK = 4
RC = R // NCK
NCH_C = RC // NW // CH


def _sc_gather_body(table_hbm, idx_hbm, out_hbm, idx_v, rows_v, sem):
    wid = _worker_id()
    base = wid * NCH_C
    pltpu.sync_copy(idx_hbm.at[pl.ds(base, NCH_C)], idx_v)

    def body(j, carry):
        pltpu.async_copy(table_hbm.at[idx_v.at[j]], rows_v, sem).wait()
        pltpu.sync_copy(rows_v, out_hbm.at[pl.ds((base + j) * CH, CH)])
        return carry

    lax.fori_loop(0, NCH_C, body, 0)


def _sc_gather(table, idx2d):
    k = pl.kernel(
        _sc_gather_body,
        out_type=jax.ShapeDtypeStruct((RC, E2), jnp.float32),
        mesh=_vmesh(),
        scratch_types=[
            pltpu.VMEM((NCH_C, CH), jnp.int32),
            pltpu.VMEM((CH, E2), jnp.float32),
            pltpu.SemaphoreType.DMA,
        ],
    )
    return k(table, idx2d)


def _mlp_body(flat_ref, body0_ref, maskf_ref, w1_ref, b1_ref, w2_ref, b2_ref,
              out_ref):
    a = (body0_ref[...] != 0).astype(jnp.float32)
    x = jnp.concatenate(
        [flat_ref[s] * a[:, s // 2:s // 2 + 1] for s in range(2 * M)],
        axis=1)
    acc = jnp.dot(x, w1_ref[...],
                  preferred_element_type=jnp.float32) + b1_ref[...]
    h = jnp.maximum(acc, 0.0)
    s = jnp.dot(h, w2_ref[...], preferred_element_type=jnp.float32)
    s = s + b2_ref[...]
    s = s + (maskf_ref[...] - 1.0) * 1000000000.0
    out_ref[...] = s.reshape(1, 1, TG)


def _mlp(flat_s, body0, maskf, W1, b1, W2, b2, k):
    bc = B // NCK
    return pl.pallas_call(
        _mlp_body,
        grid=(bc,),
        in_specs=[
            pl.BlockSpec((2 * M, TG, E2), lambda i: (0, i, 0)),
            pl.BlockSpec((TG, M), lambda i, k=k: (bc * k + i, 0)),
            pl.BlockSpec((TG, 1), lambda i, k=k: (bc * k + i, 0)),
            pl.BlockSpec((IN, H), lambda i: (0, 0)),
            pl.BlockSpec((1, H), lambda i: (0, 0)),
            pl.BlockSpec((H, 1), lambda i: (0, 0)),
            pl.BlockSpec((1, 1), lambda i: (0, 0)),
        ],
        out_specs=pl.BlockSpec((1, 1, TG), lambda i: (i, 0, 0)),
        out_shape=jax.ShapeDtypeStruct((bc, 1, TG), jnp.float32),
    )(flat_s, body0, maskf, W1, b1, W2, b2)


def _topk_body(s_ref, score_ref, gidx_ref):
    key = s_ref[...]
    l = lax.broadcasted_iota(jnp.int32, (B, TG), 1)
    idx = l
    for k in range(1, 12):
        for j in range(k - 1, -1, -1):
            d = 1 << j
            partner_hi = (l & d) != 0
            pk = jnp.where(partner_hi, pltpu.roll(key, d, 1),
                           pltpu.roll(key, TG - d, 1))
            pi = jnp.where(partner_hi, pltpu.roll(idx, d, 1),
                           pltpu.roll(idx, TG - d, 1))
            dir_desc = (l & (1 << k)) == 0
            keep_max = jnp.logical_xor(dir_desc, partner_hi)
            i_larger = (key > pk) | ((key == pk) & (idx < pi))
            take_self = keep_max == i_larger
            key = jnp.where(take_self, key, pk)
            idx = jnp.where(take_self, idx, pi)
    row = lax.broadcasted_iota(jnp.int32, (B, K_OUT), 0)
    score_ref[...] = key[:, :K_OUT]
    gidx_ref[...] = idx[:, :K_OUT] + TG * row


def _topk(scores):
    return pl.pallas_call(
        _topk_body,
        out_shape=(
            jax.ShapeDtypeStruct((B, K_OUT), jnp.float32),
            jax.ShapeDtypeStruct((B, K_OUT), jnp.int32),
        ),
    )(scores)


def _sc_select_body(pay_hbm, idx_hbm, out_hbm, idx_v, rows_v, sem):
    wid = _worker_id()
    nch = SPW // CH
    base = wid * nch
    pltpu.sync_copy(idx_hbm.at[pl.ds(base, nch)], idx_v)

    def body(j, carry):
        pltpu.async_copy(pay_hbm.at[idx_v.at[j]], rows_v, sem).wait()
        pltpu.sync_copy(rows_v, out_hbm.at[pl.ds((base + j) * CH, CH)])
        return carry

    lax.fori_loop(0, SPW // CH, body, 0)


def _sc_select(payload, idx2d):
    k = pl.kernel(
        _sc_select_body,
        out_type=jax.ShapeDtypeStruct((SEL, PW), jnp.int32),
        mesh=_vmesh(),
        scratch_types=[
            pltpu.VMEM((SPW // CH, CH), jnp.int32),
            pltpu.VMEM((CH, PW), jnp.int32),
            pltpu.SemaphoreType.DMA,
        ],
        compiler_params=pltpu.CompilerParams(use_tc_tiling_on_sc=False),
    )
    return k(payload, idx2d)


def kernel(body, mask, rule_idx, ent_table, W1, b1, W2, b2):
    gidx = jnp.transpose(body[..., 1:3], (2, 3, 0, 1))
    gidx = gidx.reshape(2 * M, NCK, RC // (2 * M))
    gidx = gidx.transpose(1, 0, 2).reshape(NCK, RC // CH, CH)

    body0 = body[..., 0].reshape(B * TG, M)
    maskf = mask.astype(jnp.float32).reshape(B * TG, 1)
    b1r, b2r = b1.reshape(1, H), b2.reshape(1, 1)

    score_chunks = []
    for k in range(NCK):
        flat = _sc_gather(ent_table, gidx[k])
        flat_s = flat.reshape(2 * M, RC // (2 * M), E2)
        score_chunks.append(
            _mlp(flat_s, body0, maskf, W1, b1r, W2, b2r, k))
    scores = jnp.concatenate(score_chunks, axis=0)

    top_scores, sel_idx = _topk(scores.reshape(B, TG))

    payload = jnp.concatenate(
        [body.reshape(B * TG, 12),
         rule_idx.reshape(B * TG, 1),
         mask.astype(jnp.int32).reshape(B * TG, 1),
         jnp.zeros((B * TG, 2), jnp.int32)], axis=1)
    sel = _sc_select(payload, sel_idx.reshape(SEL // CH, CH))
    sel = sel.reshape(B, K_OUT, PW)

    body_out = sel[..., :12].reshape(B, K_OUT, M, 3)
    rule_out = sel[..., 12]
    mask_out = sel[..., 13].astype(jnp.bool_)
    return body_out, mask_out, rule_out, top_scores


# speedup vs baseline: 2.8398x; 1.0508x over previous
import jax
import jax.numpy as jnp
from jax import lax
from jax.experimental import pallas as pl
from jax.experimental.pallas import tpu as pltpu
from jax.experimental.pallas import tpu_sc as plsc

B, TG, M = 16, 2048, 4
E2 = 128
IN = M * 2 * E2
H = IN // 2
K_OUT = 1024
NC, NS = 2, 16
NW = NC * NS
R = B * TG * M * 2
RPW = R // NW
CH = 128
NCHUNK = RPW // CH
SEL = B * K_OUT
SPW = SEL // NW
PW = 16

def _vmesh():
    return plsc.VectorSubcoreMesh(core_axis_name="c", subcore_axis_name="s",
                                  num_cores=NC, num_subcores=NS)


def _worker_id():
    return lax.axis_index("s") * NC + lax.axis_index("c")


NCK = 4
RC = R // NCK
NCH_C = RC // NW // CH


def _sc_gather_body(table_hbm, idx_hbm, out_hbm, idx_v, rows_v, sem):
    wid = _worker_id()
    base = wid * NCH_C
    pltpu.sync_copy(idx_hbm.at[pl.ds(base, NCH_C)], idx_v)
    pltpu.async_copy(table_hbm.at[idx_v.at[0]], rows_v.at[0], sem)

    def body(j, carry):
        cur = lax.rem(j, 2)
        pltpu.make_async_copy(
            table_hbm.at[idx_v.at[0]], rows_v.at[cur], sem).wait()

        @pl.when(j + 1 < NCH_C)
        def _():
            pltpu.async_copy(
                table_hbm.at[idx_v.at[j + 1]], rows_v.at[1 - cur], sem)

        pltpu.sync_copy(rows_v.at[cur], out_hbm.at[pl.ds((base + j) * CH, CH)])
        return carry

    lax.fori_loop(0, NCH_C, body, 0)


def _sc_gather(table, idx2d):
    k = pl.kernel(
        _sc_gather_body,
        out_type=jax.ShapeDtypeStruct((RC, E2), jnp.float32),
        mesh=_vmesh(),
        scratch_types=[
            pltpu.VMEM((NCH_C, CH), jnp.int32),
            pltpu.VMEM((2, CH, E2), jnp.float32),
            pltpu.SemaphoreType.DMA,
        ],
    )
    return k(table, idx2d)


def _mlp_body(flat_ref, body0_ref, maskf_ref, w1_ref, b1_ref, w2_ref, b2_ref,
              out_ref):
    a = (body0_ref[...] != 0).astype(jnp.float32)
    x = jnp.concatenate(
        [flat_ref[s] * a[:, s // 2:s // 2 + 1] for s in range(2 * M)],
        axis=1)
    acc = jnp.dot(x, w1_ref[...],
                  preferred_element_type=jnp.float32) + b1_ref[...]
    h = jnp.maximum(acc, 0.0)
    s = jnp.dot(h, w2_ref[...], preferred_element_type=jnp.float32)
    s = s + b2_ref[...]
    s = s + (maskf_ref[...] - 1.0) * 1000000000.0
    out_ref[...] = s.reshape(1, 1, TG)


def _mlp(flat_s, body0, maskf, W1, b1, W2, b2, k):
    bc = B // NCK
    return pl.pallas_call(
        _mlp_body,
        grid=(bc,),
        in_specs=[
            pl.BlockSpec((2 * M, TG, E2), lambda i: (0, i, 0)),
            pl.BlockSpec((TG, M), lambda i, k=k: (bc * k + i, 0)),
            pl.BlockSpec((TG, 1), lambda i, k=k: (bc * k + i, 0)),
            pl.BlockSpec((IN, H), lambda i: (0, 0)),
            pl.BlockSpec((1, H), lambda i: (0, 0)),
            pl.BlockSpec((H, 1), lambda i: (0, 0)),
            pl.BlockSpec((1, 1), lambda i: (0, 0)),
        ],
        out_specs=pl.BlockSpec((1, 1, TG), lambda i: (i, 0, 0)),
        out_shape=jax.ShapeDtypeStruct((bc, 1, TG), jnp.float32),
    )(flat_s, body0, maskf, W1, b1, W2, b2)


def _topk_body(s_ref, score_ref, gidx_ref):
    key = s_ref[...]
    l = lax.broadcasted_iota(jnp.int32, (B, TG), 1)
    idx = l
    for k in range(1, 12):
        for j in range(k - 1, -1, -1):
            d = 1 << j
            partner_hi = (l & d) != 0
            pk = jnp.where(partner_hi, pltpu.roll(key, d, 1),
                           pltpu.roll(key, TG - d, 1))
            pi = jnp.where(partner_hi, pltpu.roll(idx, d, 1),
                           pltpu.roll(idx, TG - d, 1))
            dir_desc = (l & (1 << k)) == 0
            keep_max = jnp.logical_xor(dir_desc, partner_hi)
            i_larger = (key > pk) | ((key == pk) & (idx < pi))
            take_self = keep_max == i_larger
            key = jnp.where(take_self, key, pk)
            idx = jnp.where(take_self, idx, pi)
    row = lax.broadcasted_iota(jnp.int32, (B, K_OUT), 0)
    score_ref[...] = key[:, :K_OUT]
    gidx_ref[...] = idx[:, :K_OUT] + TG * row


def _topk(scores):
    return pl.pallas_call(
        _topk_body,
        out_shape=(
            jax.ShapeDtypeStruct((B, K_OUT), jnp.float32),
            jax.ShapeDtypeStruct((B, K_OUT), jnp.int32),
        ),
    )(scores)


def _sc_select_body(pay_hbm, idx_hbm, out_hbm, idx_v, rows_v, sem):
    wid = _worker_id()
    nch = SPW // CH
    base = wid * nch
    pltpu.sync_copy(idx_hbm.at[pl.ds(base, nch)], idx_v)

    def body(j, carry):
        pltpu.async_copy(pay_hbm.at[idx_v.at[j]], rows_v, sem).wait()
        pltpu.sync_copy(rows_v, out_hbm.at[pl.ds((base + j) * CH, CH)])
        return carry

    lax.fori_loop(0, SPW // CH, body, 0)


def _sc_select(payload, idx2d):
    k = pl.kernel(
        _sc_select_body,
        out_type=jax.ShapeDtypeStruct((SEL, PW), jnp.int32),
        mesh=_vmesh(),
        scratch_types=[
            pltpu.VMEM((SPW // CH, CH), jnp.int32),
            pltpu.VMEM((CH, PW), jnp.int32),
            pltpu.SemaphoreType.DMA,
        ],
        compiler_params=pltpu.CompilerParams(use_tc_tiling_on_sc=False),
    )
    return k(payload, idx2d)


def kernel(body, mask, rule_idx, ent_table, W1, b1, W2, b2):
    gidx = jnp.transpose(body[..., 1:3], (2, 3, 0, 1))
    gidx = gidx.reshape(2 * M, NCK, RC // (2 * M))
    gidx = gidx.transpose(1, 0, 2).reshape(NCK, RC // CH, CH)

    body0 = body[..., 0].reshape(B * TG, M)
    maskf = mask.astype(jnp.float32).reshape(B * TG, 1)
    b1r, b2r = b1.reshape(1, H), b2.reshape(1, 1)

    score_chunks = []
    for k in range(NCK):
        flat = _sc_gather(ent_table, gidx[k])
        flat_s = flat.reshape(2 * M, RC // (2 * M), E2)
        score_chunks.append(
            _mlp(flat_s, body0, maskf, W1, b1r, W2, b2r, k))
    scores = jnp.concatenate(score_chunks, axis=0)

    top_scores, sel_idx = _topk(scores.reshape(B, TG))

    payload = jnp.concatenate(
        [body.reshape(B * TG, 12),
         rule_idx.reshape(B * TG, 1),
         mask.astype(jnp.int32).reshape(B * TG, 1),
         jnp.zeros((B * TG, 2), jnp.int32)], axis=1)
    sel = _sc_select(payload, sel_idx.reshape(SEL // CH, CH))
    sel = sel.reshape(B, K_OUT, PW)

    body_out = sel[..., :12].reshape(B, K_OUT, M, 3)
    rule_out = sel[..., 12]
    mask_out = sel[..., 13].astype(jnp.bool_)
    return body_out, mask_out, rule_out, top_scores


# speedup vs baseline: 3.1885x; 1.1228x over previous
import jax
import jax.numpy as jnp
from jax import lax
from jax.experimental import pallas as pl
from jax.experimental.pallas import tpu as pltpu
from jax.experimental.pallas import tpu_sc as plsc

B, TG, M = 16, 2048, 4
E2 = 128
IN = M * 2 * E2
H = IN // 2
K_OUT = 1024
NC, NS = 2, 16
NW = NC * NS
R = B * TG * M * 2
RPW = R // NW
CH = 128
NCHUNK = RPW // CH
SEL = B * K_OUT
SPW = SEL // NW
PW = 16

def _vmesh():
    return plsc.VectorSubcoreMesh(core_axis_name="c", subcore_axis_name="s",
                                  num_cores=NC, num_subcores=NS)


def _worker_id():
    return lax.axis_index("s") * NC + lax.axis_index("c")


NCK = 4
RC = R // NCK
NCH_C = RC // NW // CH


def _sc_gather_body(table_hbm, idx_hbm, out_hbm, idx_v, rows_v, sem):
    wid = _worker_id()
    base = wid * NCH_C
    pltpu.sync_copy(idx_hbm.at[pl.ds(base, NCH_C)], idx_v)
    pltpu.async_copy(table_hbm.at[idx_v.at[0]], rows_v.at[0], sem)

    def body(j, carry):
        cur = lax.rem(j, 2)
        pltpu.make_async_copy(
            table_hbm.at[idx_v.at[0]], rows_v.at[cur], sem).wait()

        @pl.when(j + 1 < NCH_C)
        def _():
            pltpu.async_copy(
                table_hbm.at[idx_v.at[j + 1]], rows_v.at[1 - cur], sem)

        pltpu.sync_copy(rows_v.at[cur], out_hbm.at[pl.ds((base + j) * CH, CH)])
        return carry

    lax.fori_loop(0, NCH_C, body, 0)


def _sc_gather(table, idx2d):
    k = pl.kernel(
        _sc_gather_body,
        out_type=jax.ShapeDtypeStruct((RC, E2), jnp.float32),
        mesh=_vmesh(),
        scratch_types=[
            pltpu.VMEM((NCH_C, CH), jnp.int32),
            pltpu.VMEM((4, CH, E2), jnp.float32),
            pltpu.SemaphoreType.DMA,
        ],
    )
    return k(table, idx2d)


def _mlp_body(flat_ref, body0_ref, maskf_ref, w1_ref, b1_ref, w2_ref, b2_ref,
              out_ref):
    a = (body0_ref[...] != 0).astype(jnp.float32)
    x = jnp.concatenate(
        [flat_ref[s] * a[:, s // 2:s // 2 + 1] for s in range(2 * M)],
        axis=1)
    acc = jnp.dot(x, w1_ref[...],
                  preferred_element_type=jnp.float32) + b1_ref[...]
    h = jnp.maximum(acc, 0.0)
    s = jnp.dot(h, w2_ref[...], preferred_element_type=jnp.float32)
    s = s + b2_ref[...]
    s = s + (maskf_ref[...] - 1.0) * 1000000000.0
    out_ref[...] = s.reshape(1, 1, TG)


def _mlp(flat_s, body0, maskf, W1, b1, W2, b2, k):
    bc = B // NCK
    return pl.pallas_call(
        _mlp_body,
        grid=(bc,),
        in_specs=[
            pl.BlockSpec((2 * M, TG, E2), lambda i: (0, i, 0)),
            pl.BlockSpec((TG, M), lambda i, k=k: (bc * k + i, 0)),
            pl.BlockSpec((TG, 1), lambda i, k=k: (bc * k + i, 0)),
            pl.BlockSpec((IN, H), lambda i: (0, 0)),
            pl.BlockSpec((1, H), lambda i: (0, 0)),
            pl.BlockSpec((H, 1), lambda i: (0, 0)),
            pl.BlockSpec((1, 1), lambda i: (0, 0)),
        ],
        out_specs=pl.BlockSpec((1, 1, TG), lambda i: (i, 0, 0)),
        out_shape=jax.ShapeDtypeStruct((bc, 1, TG), jnp.float32),
    )(flat_s, body0, maskf, W1, b1, W2, b2)


def _cmp_exchange(key, idx, l, d, n, keep_max):
    partner_hi = (l & d) != 0
    pk = jnp.where(partner_hi, pltpu.roll(key, d, 1),
                   pltpu.roll(key, n - d, 1))
    pi = jnp.where(partner_hi, pltpu.roll(idx, d, 1),
                   pltpu.roll(idx, n - d, 1))
    km = jnp.logical_xor(keep_max, partner_hi)
    i_larger = (key > pk) | ((key == pk) & (idx < pi))
    take_self = km == i_larger
    return jnp.where(take_self, key, pk), jnp.where(take_self, idx, pi)


def _topk_body(s_ref, score_ref, gidx_ref):
    key = s_ref[...]
    l = lax.broadcasted_iota(jnp.int32, (B, TG), 1)
    idx = l
    true2 = jnp.full((B, TG), True)
    for k in range(1, 11):
        for j in range(k - 1, -1, -1):
            key, idx = _cmp_exchange(key, idx, l, 1 << j, TG,
                                     (l & (1 << k)) == 0)
    key, idx = _cmp_exchange(key, idx, l, K_OUT, TG, true2)
    key, idx, l = key[:, :K_OUT], idx[:, :K_OUT], l[:, :K_OUT]
    true1 = jnp.full((B, K_OUT), True)
    for j in range(9, -1, -1):
        key, idx = _cmp_exchange(key, idx, l, 1 << j, K_OUT, true1)
    row = lax.broadcasted_iota(jnp.int32, (B, K_OUT), 0)
    score_ref[...] = key
    gidx_ref[...] = idx + TG * row


def _topk(scores):
    return pl.pallas_call(
        _topk_body,
        out_shape=(
            jax.ShapeDtypeStruct((B, K_OUT), jnp.float32),
            jax.ShapeDtypeStruct((B, K_OUT), jnp.int32),
        ),
    )(scores)


NF = 14


def _sc_select_body(pay_hbm, idx_hbm, out_hbm, idx_v, planes_v, sem):
    wid = _worker_id()
    nch = SPW // CH
    base = wid * nch
    pltpu.sync_copy(idx_hbm.at[pl.ds(base, nch)], idx_v)

    for j in range(nch):
        for f in range(NF):
            pltpu.async_copy(pay_hbm.at[f].at[idx_v.at[j]],
                             planes_v.at[f, pl.ds(j * CH, CH)], sem)
        pltpu.make_async_copy(pay_hbm.at[:, pl.ds(0, CH)],
                              planes_v.at[:, pl.ds(j * CH, CH)], sem).wait()

    pltpu.sync_copy(planes_v, out_hbm.at[:, pl.ds(wid * SPW, SPW)])


def _sc_select(pay_soa, idx2d):
    k = pl.kernel(
        _sc_select_body,
        out_type=jax.ShapeDtypeStruct((NF, SEL), jnp.int32),
        mesh=_vmesh(),
        scratch_types=[
            pltpu.VMEM((SPW // CH, CH), jnp.int32),
            pltpu.VMEM((NF, SPW), jnp.int32),
            pltpu.SemaphoreType.DMA,
        ],
        compiler_params=pltpu.CompilerParams(use_tc_tiling_on_sc=False),
    )
    return k(pay_soa, idx2d)


def kernel(body, mask, rule_idx, ent_table, W1, b1, W2, b2):
    gidx = jnp.transpose(body[..., 1:3], (2, 3, 0, 1))
    gidx = gidx.reshape(2 * M, NCK, RC // (2 * M))
    gidx = gidx.transpose(1, 0, 2).reshape(NCK, RC // CH, CH)

    body0 = body[..., 0].reshape(B * TG, M)
    maskf = mask.astype(jnp.float32).reshape(B * TG, 1)
    b1r, b2r = b1.reshape(1, H), b2.reshape(1, 1)

    score_chunks = []
    for k in range(NCK):
        flat = _sc_gather(ent_table, gidx[k])
        flat_s = flat.reshape(2 * M, RC // (2 * M), E2)
        score_chunks.append(
            _mlp(flat_s, body0, maskf, W1, b1r, W2, b2r, k))
    scores = jnp.concatenate(score_chunks, axis=0)

    top_scores, sel_idx = _topk(scores.reshape(B, TG))

    pay_soa = jnp.concatenate(
        [jnp.transpose(body, (2, 3, 0, 1)).reshape(12, B * TG),
         rule_idx.reshape(1, B * TG),
         mask.astype(jnp.int32).reshape(1, B * TG)], axis=0)
    sel = _sc_select(pay_soa, sel_idx.reshape(SEL // CH, CH))

    body_out = sel[:12].reshape(M, 3, B, K_OUT).transpose(2, 3, 0, 1)
    rule_out = sel[12].reshape(B, K_OUT)
    mask_out = sel[13].reshape(B, K_OUT).astype(jnp.bool_)
    return body_out, mask_out, rule_out, top_scores
